# Initial kernel scaffold; baseline (speedup 1.0000x reference)
import jax
import jax.numpy as jnp
from jax.experimental import pallas as pl


def kernel(inp, plane0, plane1, plane2):
    raise NotImplementedError("write your pallas kernel here")


# speedup vs baseline: 47.2560x; 47.2560x over previous
import functools

import jax
import jax.numpy as jnp
from jax.experimental import pallas as pl
from jax.experimental.pallas import tpu as pltpu
from jax.experimental.pallas import tpu_sc as plsc

_COMBS = ((0, 3), (1, 3), (2, 3))
_WINDOW = 256
_BLK = 1024


def _pack_plane(plane):
    t = jnp.transpose(plane, (1, 2, 0))
    tx = jnp.concatenate([t[:, 1:], t[:, -1:]], axis=1)
    ty = jnp.concatenate([t[1:], t[-1:]], axis=0)
    txy = jnp.concatenate([tx[1:], tx[-1:]], axis=0)
    H, W, C = t.shape
    return jnp.concatenate([t, tx, ty, txy], axis=-1).reshape(H * W, 4 * C)


def _cell_index(inp, plane_shape, comb):
    _, H, W = plane_shape
    x = jnp.clip((inp[:, comb[0]] + 1.0) * (0.5 * (W - 1)), 0.0, W - 1)
    y = jnp.clip((inp[:, comb[1]] + 1.0) * (0.5 * (H - 1)), 0.0, H - 1)
    return jnp.floor(y).astype(jnp.int32) * W + jnp.floor(x).astype(jnp.int32)


def _sc_gather(table, idx):
    n = idx.shape[0]
    d = table.shape[1]
    mesh = plsc.VectorSubcoreMesh(core_axis_name="c", subcore_axis_name="s")

    @functools.partial(
        pl.kernel,
        out_type=jax.ShapeDtypeStruct((n, d), table.dtype),
        mesh=mesh,
    )
    def gather_kernel(table_hbm, idx_hbm, out_hbm):
        def body(i_vmem, o_vmem):
            pltpu.sync_copy(table_hbm.at[i_vmem.at[0]], o_vmem)

        pltpu.emit_pipeline(
            body,
            grid=(n // _WINDOW,),
            in_specs=[pl.BlockSpec((1, _WINDOW), lambda i: (0, i))],
            out_specs=[pl.BlockSpec((_WINDOW, d), lambda i: (i, 0))],
            core_axis_name=("c", "s"),
            dimension_semantics=(pltpu.PARALLEL,),
        )(idx_hbm, out_hbm)

    return gather_kernel(table, idx.reshape(1, n))


def _combine(inp, g3, plane_shapes):
    P = inp.shape[0]
    C = g3.shape[2] // 4

    def body(inp_ref, g_ref, o_ref):
        acc = jnp.zeros((_BLK, C), jnp.float32)
        for ci, comb in enumerate(_COMBS):
            _, H, W = plane_shapes[ci]
            cx = inp_ref[:, comb[0]:comb[0] + 1]
            cy = inp_ref[:, comb[1]:comb[1] + 1]
            x = jnp.clip((cx + 1.0) * (0.5 * (W - 1)), 0.0, W - 1)
            y = jnp.clip((cy + 1.0) * (0.5 * (H - 1)), 0.0, H - 1)
            wx = x - jnp.floor(x)
            wy = y - jnp.floor(y)
            g = g_ref[ci]
            acc = acc + ((g[:, 0:C] * (1.0 - wx) + g[:, C:2 * C] * wx)
                         * (1.0 - wy)
                         + (g[:, 2 * C:3 * C] * (1.0 - wx)
                            + g[:, 3 * C:4 * C] * wx) * wy)
        o_ref[...] = acc

    return pl.pallas_call(
        body,
        grid=(P // _BLK,),
        in_specs=[
            pl.BlockSpec((_BLK, inp.shape[1]), lambda i: (i, 0)),
            pl.BlockSpec((3, _BLK, 4 * C), lambda i: (0, i, 0)),
        ],
        out_specs=pl.BlockSpec((_BLK, C), lambda i: (i, 0)),
        out_shape=jax.ShapeDtypeStruct((P, C), jnp.float32),
    )(inp, g3)


def kernel(inp, plane0, plane1, plane2):
    planes = (plane0, plane1, plane2)
    P = inp.shape[0]
    tables = [_pack_plane(p) for p in planes]
    offs = []
    base = 0
    for t in tables:
        offs.append(base)
        base += t.shape[0]
    table = jnp.concatenate(tables, axis=0)
    idx = jnp.concatenate([
        _cell_index(inp, p.shape, c) + o
        for p, c, o in zip(planes, _COMBS, offs)
    ])
    g = _sc_gather(table, idx)
    g3 = g.reshape(len(planes), P, table.shape[1])
    return _combine(inp, g3, [p.shape for p in planes])


# speedup vs baseline: 71.9391x; 1.5223x over previous
import functools

import jax
import jax.numpy as jnp
import numpy as np
from jax.experimental import pallas as pl
from jax.experimental.pallas import tpu as pltpu
from jax.experimental.pallas import tpu_sc as plsc

_COMBS = ((0, 3), (1, 3), (2, 3))
_WINDOW = 256
_BLK = 2048


def _pack_plane(plane):
    t = jnp.transpose(plane, (1, 2, 0))
    tx = jnp.concatenate([t[:, 1:], t[:, -1:]], axis=1)
    ty = jnp.concatenate([t[1:], t[-1:]], axis=0)
    txy = jnp.concatenate([tx[1:], tx[-1:]], axis=0)
    H, W, C = t.shape
    return jnp.concatenate([t, tx, ty, txy], axis=-1).reshape(H * W, 4 * C)


def _sc_gather3(tables, idx):
    n = idx.shape[0]
    nper = n // len(tables)
    d = tables[0].shape[1]
    mesh = plsc.VectorSubcoreMesh(core_axis_name="c", subcore_axis_name="s")

    @functools.partial(
        pl.kernel,
        out_type=jax.ShapeDtypeStruct((n, d), tables[0].dtype),
        mesh=mesh,
    )
    def gather_kernel(t0_hbm, t1_hbm, t2_hbm, idx_hbm, out_hbm):
        for ci, t_hbm in enumerate((t0_hbm, t1_hbm, t2_hbm)):
            def body(i_vmem, o_vmem, t_hbm=t_hbm):
                pltpu.sync_copy(t_hbm.at[i_vmem.at[0]], o_vmem)

            base = ci * (nper // _WINDOW)
            pltpu.emit_pipeline(
                body,
                grid=(nper // _WINDOW,),
                in_specs=[pl.BlockSpec((1, _WINDOW),
                                       lambda i, base=base: (0, base + i))],
                out_specs=[pl.BlockSpec((_WINDOW, d),
                                        lambda i, base=base: (base + i, 0))],
                core_axis_name=("c", "s"),
                dimension_semantics=(pltpu.PARALLEL,),
            )(idx_hbm, out_hbm)

    return gather_kernel(*tables, idx.reshape(1, n))


def _combine(w16, g3, E, S):
    P = w16.shape[1]
    D = g3.shape[2]
    C = S.shape[1]

    def body(w_ref, g_ref, E_ref, S_ref, o_ref):
        wt = jnp.transpose(w_ref[...], (1, 0))
        wpat = jax.lax.dot_general(
            wt, E_ref[...], (((1,), (0,)), ((), ())),
            preferred_element_type=jnp.float32)
        acc = g_ref[0] * wpat[:, 0:D]
        acc = acc + g_ref[1] * wpat[:, D:2 * D]
        acc = acc + g_ref[2] * wpat[:, 2 * D:3 * D]
        o_ref[...] = jax.lax.dot_general(
            acc.astype(jnp.bfloat16), S_ref[...], (((1,), (0,)), ((), ())),
            preferred_element_type=jnp.float32)

    return pl.pallas_call(
        body,
        grid=(P // _BLK,),
        in_specs=[
            pl.BlockSpec((16, _BLK), lambda i: (0, i)),
            pl.BlockSpec((3, _BLK, D), lambda i: (0, i, 0)),
            pl.BlockSpec(E.shape, lambda i: (0, 0)),
            pl.BlockSpec(S.shape, lambda i: (0, 0)),
        ],
        out_specs=pl.BlockSpec((_BLK, C), lambda i: (i, 0)),
        out_shape=jax.ShapeDtypeStruct((P, C), jnp.float32),
    )(w16, g3, E, S)


def kernel(inp, plane0, plane1, plane2):
    planes = (plane0, plane1, plane2)
    P = inp.shape[0]
    C = plane0.shape[0]
    D = 4 * C
    tables = [_pack_plane(p) for p in planes]
    cT = inp.T
    idx_parts = []
    w_rows = []
    for ci, comb in enumerate(_COMBS):
        _, H, W = planes[ci].shape
        x = jnp.clip((cT[comb[0]] + 1.0) * (0.5 * (W - 1)), 0.0, W - 1)
        y = jnp.clip((cT[comb[1]] + 1.0) * (0.5 * (H - 1)), 0.0, H - 1)
        x0 = jnp.floor(x)
        y0 = jnp.floor(y)
        idx_parts.append(y0.astype(jnp.int32) * W + x0.astype(jnp.int32))
        wx = x - x0
        wy = y - y0
        w_rows += [(1.0 - wx) * (1.0 - wy), wx * (1.0 - wy),
                   (1.0 - wx) * wy, wx * wy]
    w_rows += [jnp.zeros((P,), jnp.float32)] * 4
    w16 = jnp.stack(w_rows).astype(jnp.bfloat16)

    E = np.zeros((16, 3 * D), np.float32)
    for ci in range(3):
        for c in range(4):
            E[4 * ci + c, ci * D + c * C:ci * D + (c + 1) * C] = 1.0
    S = np.zeros((D, C), np.float32)
    for c in range(4):
        S[c * C:(c + 1) * C, :] += np.eye(C, dtype=np.float32)
    E = jnp.asarray(E, jnp.bfloat16)
    S = jnp.asarray(S, jnp.bfloat16)

    g = _sc_gather3(tables, jnp.concatenate(idx_parts))
    return _combine(w16, g.reshape(3, P, D), E, S)
